# baseline (speedup 1.0000x reference)
import functools

import jax
import jax.numpy as jnp
from jax import lax
from jax.experimental import pallas as pl
from jax.experimental.pallas import tpu as pltpu
from jax.experimental.pallas import tpu_sc as plsc

_TM = 128


def _routing_tables(eid, n_tok, tm, n_exp):
    t = n_tok // tm
    p_max = t + n_exp - 1
    tok = jnp.arange(n_tok, dtype=jnp.int32)
    keys = eid * jnp.int32(n_tok) + tok
    perm = jnp.sort(keys) % jnp.int32(n_tok)
    inv_perm = jnp.zeros((n_tok,), jnp.int32).at[perm].set(tok)

    counts = jnp.bincount(eid, length=n_exp).astype(jnp.int32)
    ends = jnp.cumsum(counts)
    starts = ends - counts
    nonempty = counts > 0
    first = jnp.where(nonempty, starts // tm, 0)
    last = jnp.where(nonempty, (ends - 1) // tm, 0)
    span = jnp.where(nonempty, last - first + 1, 0)
    pair_start = jnp.cumsum(span) - span
    total = pair_start[n_exp - 1] + span[n_exp - 1]

    p = jnp.arange(p_max, dtype=jnp.int32)
    e_raw = (jnp.searchsorted(pair_start, p, side="right") - 1).astype(jnp.int32)
    e_raw = jnp.clip(e_raw, 0, n_exp - 1)
    tile_raw = jnp.clip(first[e_raw] + (p - pair_start[e_raw]), 0, t - 1)
    pad = p >= total
    experts = jnp.where(pad, jnp.take(e_raw, total - 1), e_raw)
    tiles = jnp.where(pad, t - 1, tile_raw)
    lo_g = jnp.clip(starts[experts], tiles * tm, (tiles + 1) * tm)
    hi_g = jnp.clip(ends[experts], tiles * tm, (tiles + 1) * tm)
    lo = lo_g - tiles * tm
    hi = hi_g - tiles * tm
    return perm, inv_perm, tiles, experts, lo, hi


def _ffn_body(tiles_r, experts_r, lo_r, hi_r,
              x_ref, w1_ref, b1_ref, w2_ref, b2_ref, o_ref):
    i = pl.program_id(0)
    lo = lo_r[i]
    hi = hi_r[i]

    @pl.when(lo < hi)
    def _():
        xb = x_ref[...].astype(jnp.bfloat16)
        w1 = w1_ref[0].astype(jnp.bfloat16)
        h = lax.dot_general(xb, w1, (((1,), (1,)), ((), ())),
                            preferred_element_type=jnp.float32)
        h = jnp.maximum(h + b1_ref[0], 0.0)
        w2 = w2_ref[0].astype(jnp.bfloat16)
        y = lax.dot_general(h.astype(jnp.bfloat16), w2, (((1,), (1,)), ((), ())),
                            preferred_element_type=jnp.float32)
        y = y + b2_ref[0]
        rows = lax.broadcasted_iota(jnp.int32, (xb.shape[0], 1), 0)
        m = (rows >= lo) & (rows < hi)
        o_ref[...] = jnp.where(m, y, o_ref[...])


def _grouped_ffn(x_sorted, w1, b1r, w2, b2r, tiles, experts, lo, hi):
    n_tok, in_dim = x_sorted.shape
    n_exp, hid, _ = w1.shape
    out_dim = w2.shape[1]
    p_max = tiles.shape[0]
    grid_spec = pltpu.PrefetchScalarGridSpec(
        num_scalar_prefetch=4,
        grid=(p_max,),
        in_specs=[
            pl.BlockSpec((_TM, in_dim), lambda i, tr, er, lr, hr: (tr[i], 0)),
            pl.BlockSpec((1, hid, in_dim), lambda i, tr, er, lr, hr: (er[i], 0, 0)),
            pl.BlockSpec((1, 1, hid), lambda i, tr, er, lr, hr: (er[i], 0, 0)),
            pl.BlockSpec((1, out_dim, hid), lambda i, tr, er, lr, hr: (er[i], 0, 0)),
            pl.BlockSpec((1, 1, out_dim), lambda i, tr, er, lr, hr: (er[i], 0, 0)),
        ],
        out_specs=pl.BlockSpec((_TM, out_dim), lambda i, tr, er, lr, hr: (tr[i], 0)),
    )
    return pl.pallas_call(
        _ffn_body,
        grid_spec=grid_spec,
        out_shape=jax.ShapeDtypeStruct((n_tok, out_dim), jnp.float32),
    )(tiles, experts, lo, hi, x_sorted, w1, b1r, w2, b2r)


@functools.lru_cache(maxsize=None)
def _make_sc_gather(n_rows, d):
    info = plsc.get_sparse_core_info()
    nc, ns = info.num_cores, info.num_subcores
    nw = nc * ns
    bpw = n_rows // nw
    mesh = plsc.VectorSubcoreMesh(core_axis_name="c", subcore_axis_name="s")

    @functools.partial(
        pl.kernel, mesh=mesh,
        out_type=jax.ShapeDtypeStruct((n_rows, d), jnp.float32),
        scratch_types=[
            pltpu.VMEM((bpw,), jnp.int32),
            pltpu.VMEM((bpw, d), jnp.float32),
            pltpu.SemaphoreType.DMA,
        ],
    )
    def gather_k(table_hbm, idx_hbm, out_hbm, idx_v, rows_v, sem):
        wid = lax.axis_index("s") * nc + lax.axis_index("c")
        base = wid * bpw
        pltpu.sync_copy(idx_hbm.at[pl.ds(base, bpw)], idx_v)
        pltpu.async_copy(table_hbm.at[idx_v], rows_v, sem).wait()
        pltpu.sync_copy(rows_v, out_hbm.at[pl.ds(base, bpw)])

    return gather_k


def kernel(x, Wg, bg, W1, b1, W2, b2):
    n_tok, in_dim = x.shape
    n_exp, hid, _ = W1.shape
    out_dim = W2.shape[1]

    gating_scores = x @ Wg.T + bg
    _, top_idx = lax.top_k(gating_scores, 1)
    eid = top_idx[:, 0].astype(jnp.int32)

    perm, inv_perm, tiles, experts, lo, hi = _routing_tables(
        eid, n_tok, _TM, n_exp)

    gather = _make_sc_gather(n_tok, in_dim)
    x_sorted = gather(x, perm)

    y_sorted = _grouped_ffn(
        x_sorted, W1, b1.reshape(n_exp, 1, hid), W2,
        b2.reshape(n_exp, 1, out_dim), tiles, experts, lo, hi)

    combine = _make_sc_gather(n_tok, out_dim)
    return combine(y_sorted, inv_perm)


# speedup vs baseline: 1.0259x; 1.0259x over previous
import functools

import jax
import jax.numpy as jnp
from jax import lax
from jax.experimental import pallas as pl
from jax.experimental.pallas import tpu as pltpu
from jax.experimental.pallas import tpu_sc as plsc

_TM = 128


def _routing_tables(eid, n_tok, tm, n_exp):
    t = n_tok // tm
    p_max = t + n_exp - 1
    tok = jnp.arange(n_tok, dtype=jnp.int32)
    keys = eid * jnp.int32(n_tok) + tok
    perm = jnp.sort(keys) % jnp.int32(n_tok)
    inv_perm = jnp.zeros((n_tok,), jnp.int32).at[perm].set(tok)

    counts = jnp.bincount(eid, length=n_exp).astype(jnp.int32)
    ends = jnp.cumsum(counts)
    starts = ends - counts
    nonempty = counts > 0
    first = jnp.where(nonempty, starts // tm, 0)
    last = jnp.where(nonempty, (ends - 1) // tm, 0)
    span = jnp.where(nonempty, last - first + 1, 0)
    pair_start = jnp.cumsum(span) - span
    total = pair_start[n_exp - 1] + span[n_exp - 1]

    p = jnp.arange(p_max, dtype=jnp.int32)
    e_raw = (jnp.searchsorted(pair_start, p, side="right") - 1).astype(jnp.int32)
    e_raw = jnp.clip(e_raw, 0, n_exp - 1)
    tile_raw = jnp.clip(first[e_raw] + (p - pair_start[e_raw]), 0, t - 1)
    pad = p >= total
    experts = jnp.where(pad, jnp.take(e_raw, total - 1), e_raw)
    tiles = jnp.where(pad, t - 1, tile_raw)
    lo_g = jnp.clip(starts[experts], tiles * tm, (tiles + 1) * tm)
    hi_g = jnp.clip(ends[experts], tiles * tm, (tiles + 1) * tm)
    lo = lo_g - tiles * tm
    hi = hi_g - tiles * tm
    return perm, inv_perm, tiles, experts, lo, hi


def _ffn_body(tiles_r, experts_r, lo_r, hi_r,
              x_ref, w1a_ref, w1b_ref, b1_ref, w2a_ref, w2b_ref, b2_ref,
              o_ref):
    i = pl.program_id(0)
    lo = lo_r[i]
    hi = hi_r[i]

    @pl.when(lo < hi)
    def _():
        xb = x_ref[...]
        nt = (((1,), (1,)), ((), ()))
        ha = lax.dot_general(xb, w1a_ref[0], nt,
                             preferred_element_type=jnp.float32)
        hb = lax.dot_general(xb, w1b_ref[0], nt,
                             preferred_element_type=jnp.float32)
        h = jnp.concatenate([ha, hb], axis=1)
        h = jnp.maximum(h + b1_ref[0], 0.0)
        ya = lax.dot_general(h, w2a_ref[0], nt,
                             preferred_element_type=jnp.float32)
        yb = lax.dot_general(h, w2b_ref[0], nt,
                             preferred_element_type=jnp.float32)
        y = jnp.concatenate([ya, yb], axis=1)
        y = y + b2_ref[0]
        rows = lax.broadcasted_iota(jnp.int32, (xb.shape[0], 1), 0)
        m = (rows >= lo) & (rows < hi)
        o_ref[...] = jnp.where(m, y, o_ref[...])


def _grouped_ffn(x_sorted, w1, b1r, w2, b2r, tiles, experts, lo, hi):
    n_tok, in_dim = x_sorted.shape
    n_exp, hid, _ = w1.shape
    out_dim = w2.shape[1]
    p_max = tiles.shape[0]
    hh = hid // 2
    oh = out_dim // 2
    grid_spec = pltpu.PrefetchScalarGridSpec(
        num_scalar_prefetch=4,
        grid=(p_max,),
        in_specs=[
            pl.BlockSpec((_TM, in_dim), lambda i, tr, er, lr, hr: (tr[i], 0)),
            pl.BlockSpec((1, hh, in_dim), lambda i, tr, er, lr, hr: (er[i], 0, 0)),
            pl.BlockSpec((1, hh, in_dim), lambda i, tr, er, lr, hr: (er[i], 1, 0)),
            pl.BlockSpec((1, 1, hid), lambda i, tr, er, lr, hr: (er[i], 0, 0)),
            pl.BlockSpec((1, oh, hid), lambda i, tr, er, lr, hr: (er[i], 0, 0)),
            pl.BlockSpec((1, oh, hid), lambda i, tr, er, lr, hr: (er[i], 1, 0)),
            pl.BlockSpec((1, 1, out_dim), lambda i, tr, er, lr, hr: (er[i], 0, 0)),
        ],
        out_specs=pl.BlockSpec((_TM, out_dim), lambda i, tr, er, lr, hr: (tr[i], 0)),
    )
    return pl.pallas_call(
        _ffn_body,
        grid_spec=grid_spec,
        out_shape=jax.ShapeDtypeStruct((n_tok, out_dim), jnp.float32),
    )(tiles, experts, lo, hi, x_sorted, w1, w1, b1r, w2, w2, b2r)


@functools.lru_cache(maxsize=None)
def _make_sc_gather(n_rows, d):
    info = plsc.get_sparse_core_info()
    nc, ns = info.num_cores, info.num_subcores
    nw = nc * ns
    bpw = n_rows // nw
    mesh = plsc.VectorSubcoreMesh(core_axis_name="c", subcore_axis_name="s")

    @functools.partial(
        pl.kernel, mesh=mesh,
        out_type=jax.ShapeDtypeStruct((n_rows, d), jnp.float32),
        scratch_types=[
            pltpu.VMEM((bpw,), jnp.int32),
            pltpu.VMEM((bpw, d), jnp.float32),
            pltpu.SemaphoreType.DMA,
        ],
    )
    def gather_k(table_hbm, idx_hbm, out_hbm, idx_v, rows_v, sem):
        wid = lax.axis_index("s") * nc + lax.axis_index("c")
        base = wid * bpw
        pltpu.sync_copy(idx_hbm.at[pl.ds(base, bpw)], idx_v)
        pltpu.async_copy(table_hbm.at[idx_v], rows_v, sem).wait()
        pltpu.sync_copy(rows_v, out_hbm.at[pl.ds(base, bpw)])

    return gather_k


def kernel(x, Wg, bg, W1, b1, W2, b2):
    n_tok, in_dim = x.shape
    n_exp, hid, _ = W1.shape
    out_dim = W2.shape[1]

    gating_scores = x @ Wg.T + bg
    _, top_idx = lax.top_k(gating_scores, 1)
    eid = top_idx[:, 0].astype(jnp.int32)

    perm, inv_perm, tiles, experts, lo, hi = _routing_tables(
        eid, n_tok, _TM, n_exp)

    gather = _make_sc_gather(n_tok, in_dim)
    x_sorted = gather(x, perm)

    y_sorted = _grouped_ffn(
        x_sorted, W1, b1.reshape(n_exp, 1, hid), W2,
        b2.reshape(n_exp, 1, out_dim), tiles, experts, lo, hi)

    combine = _make_sc_gather(n_tok, out_dim)
    return combine(y_sorted, inv_perm)
